# Initial kernel scaffold; baseline (speedup 1.0000x reference)
import jax
import jax.numpy as jnp
from jax.experimental import pallas as pl


def kernel(uid_batch_ph, mid_batch_ph, cat_batch_ph, mid_his_batch_ph, cat_his_batch_ph, uid_table, mid_table, cat_table, bn_gamma, bn_beta, W1, b1, alpha1, W2, b2, alpha2, W3, b3):
    raise NotImplementedError("write your pallas kernel here")


# speedup vs baseline: 7.7352x; 7.7352x over previous
import functools
import math

import jax
import jax.numpy as jnp
from jax import lax
from jax.experimental import pallas as pl
from jax.experimental.pallas import tpu as pltpu
from jax.experimental.pallas import tpu_sc as plsc

B = 4096
HIST = 50
EDIM = 64
NC, NS = 2, 16
NW = NC * NS
PW = B // NW
CB = 8
NCH = PW // CB
GCH = 80
NG = (CB * HIST) // GCH

_f32 = jnp.float32
_i32 = jnp.int32


def _sc_body(uid_i, mid_i, cat_i, mh_i, ch_i, uid_t, mid_t, cat_t,
             uid_o, mid_o, cat_o, ms_o, cs_o,
             uidv, midv, catv, mhv, chv,
             uid_r, mid_r, cat_r, mh_r, ch_r, ms_v, cs_v, sem):
    wid = lax.axis_index("s") * NC + lax.axis_index("c")

    def chunk(c, carry):
        base = wid * PW + c * CB
        b50 = base * HIST
        pltpu.sync_copy(uid_i.at[pl.ds(base, CB)], uidv)
        pltpu.sync_copy(mid_i.at[pl.ds(base, CB)], midv)
        pltpu.sync_copy(cat_i.at[pl.ds(base, CB)], catv)
        pltpu.sync_copy(mh_i.at[pl.ds(b50, CB * HIST)], mhv)
        pltpu.sync_copy(ch_i.at[pl.ds(b50, CB * HIST)], chv)
        cps = [
            pltpu.async_copy(uid_t.at[uidv], uid_r, sem),
            pltpu.async_copy(mid_t.at[midv], mid_r, sem),
            pltpu.async_copy(cat_t.at[catv], cat_r, sem),
        ]
        for g in range(NG):
            s = pl.ds(g * GCH, GCH)
            cps.append(pltpu.async_copy(mid_t.at[mhv.at[s]], mh_r.at[s], sem))
            cps.append(pltpu.async_copy(cat_t.at[chv.at[s]], ch_r.at[s], sem))
        for cp in cps:
            cp.wait()
        for b in range(CB):
            def red(rows):
                def body(j, acc):
                    r = b * HIST + j
                    return tuple(acc[d] + rows[r, pl.ds(16 * d, 16)]
                                 for d in range(4))
                z = jnp.zeros((16,), _f32)
                return lax.fori_loop(0, HIST, body, (z, z, z, z))
            am = red(mh_r)
            ac = red(ch_r)
            for d in range(4):
                ms_v[b, pl.ds(16 * d, 16)] = am[d]
                cs_v[b, pl.ds(16 * d, 16)] = ac[d]
        dst = pl.ds(base, CB)
        pltpu.sync_copy(uid_r, uid_o.at[dst])
        pltpu.sync_copy(mid_r, mid_o.at[dst])
        pltpu.sync_copy(cat_r, cat_o.at[dst])
        pltpu.sync_copy(ms_v, ms_o.at[dst])
        pltpu.sync_copy(cs_v, cs_o.at[dst])
        return carry

    lax.fori_loop(0, NCH, chunk, 0)


_sc_embed = functools.partial(
    pl.kernel,
    out_type=[jax.ShapeDtypeStruct((B, EDIM), _f32)] * 5,
    mesh=plsc.VectorSubcoreMesh(core_axis_name="c", subcore_axis_name="s"),
    compiler_params=pltpu.CompilerParams(use_tc_tiling_on_sc=False),
    scratch_types=[
        pltpu.VMEM((CB,), _i32),
        pltpu.VMEM((CB,), _i32),
        pltpu.VMEM((CB,), _i32),
        pltpu.VMEM((CB * HIST,), _i32),
        pltpu.VMEM((CB * HIST,), _i32),
        pltpu.VMEM((CB, EDIM), _f32),
        pltpu.VMEM((CB, EDIM), _f32),
        pltpu.VMEM((CB, EDIM), _f32),
        pltpu.VMEM((CB * HIST, EDIM), _f32),
        pltpu.VMEM((CB * HIST, EDIM), _f32),
        pltpu.VMEM((CB, EDIM), _f32),
        pltpu.VMEM((CB, EDIM), _f32),
        pltpu.SemaphoreType.DMA,
    ],
)(_sc_body)


_BT = 1024
_BN_SCALE = 1.0 / math.sqrt(1.0 + 1e-3)


def _mlp_body(u, m, c, ms, cs, g, bt, W1, b1, a1, W2, b2, a2, W3, b3, out):
    x = jnp.concatenate([u[...], m[...], c[...], ms[...], cs[...]], axis=1)
    x = g[...] * (x * _BN_SCALE) + bt[...]
    h = jnp.dot(x, W1[...], preferred_element_type=_f32,
                precision=lax.Precision.HIGHEST) + b1[...]
    h = jnp.maximum(h, 0.0) + a1[...] * jnp.minimum(h, 0.0)
    h = jnp.dot(h, W2[...], preferred_element_type=_f32,
                precision=lax.Precision.HIGHEST) + b2[...]
    h = jnp.maximum(h, 0.0) + a2[...] * jnp.minimum(h, 0.0)
    z = jnp.dot(h, W3[...], preferred_element_type=_f32,
                precision=lax.Precision.HIGHEST) + b3[...]
    z = z - jnp.max(z, axis=1, keepdims=True)
    e = jnp.exp(z)
    out[...] = e / jnp.sum(e, axis=1, keepdims=True) + 1e-8


def _mlp(u, m, c, ms, cs, g, bt, W1, b1, a1, W2, b2, a2, W3, b3):
    emb_spec = pl.BlockSpec((_BT, EDIM), lambda i: (i, 0))
    def full(arr):
        return pl.BlockSpec(arr.shape, lambda i: (0,) * arr.ndim)
    return pl.pallas_call(
        _mlp_body,
        grid=(B // _BT,),
        in_specs=[emb_spec] * 5 + [full(a) for a in
                                   (g, bt, W1, b1, a1, W2, b2, a2, W3, b3)],
        out_specs=pl.BlockSpec((_BT, 2), lambda i: (i, 0)),
        out_shape=jax.ShapeDtypeStruct((B, 2), _f32),
    )(u, m, c, ms, cs, g, bt, W1, b1, a1, W2, b2, a2, W3, b3)


def kernel(uid_batch_ph, mid_batch_ph, cat_batch_ph, mid_his_batch_ph,
           cat_his_batch_ph, uid_table, mid_table, cat_table, bn_gamma,
           bn_beta, W1, b1, alpha1, W2, b2, alpha2, W3, b3):
    uid = uid_batch_ph.astype(_i32)
    mid = mid_batch_ph.astype(_i32)
    cat = cat_batch_ph.astype(_i32)
    mh = mid_his_batch_ph.astype(_i32).reshape(-1)
    ch = cat_his_batch_ph.astype(_i32).reshape(-1)
    ue, me, ce, msum, csum = _sc_embed(uid, mid, cat, mh, ch,
                                       uid_table, mid_table, cat_table)
    r2 = lambda a: a.reshape(1, -1)
    return _mlp(ue, me, ce, msum, csum, r2(bn_gamma), r2(bn_beta),
                W1, r2(b1), r2(alpha1), W2, r2(b2), r2(alpha2),
                W3, r2(b3))


# speedup vs baseline: 8.5999x; 1.1118x over previous
import functools
import math

import jax
import jax.numpy as jnp
from jax import lax
from jax.experimental import pallas as pl
from jax.experimental.pallas import tpu as pltpu
from jax.experimental.pallas import tpu_sc as plsc

B = 4096
HIST = 50
EDIM = 64
NC, NS = 2, 16
NW = NC * NS
PW = B // NW
CB = 8
NCH = PW // CB
GCH = 80
NG = (CB * HIST) // GCH
_UNR = 5

_f32 = jnp.float32
_bf16 = jnp.bfloat16
_i32 = jnp.int32


def _hist_pass(tab, idxv, hb0, hb1, sem0, sem1, sum_all):
    def fire(c, buf, sem):
        for g in range(NG):
            src = tab.at[idxv.at[pl.ds(c * CB * HIST + g * GCH, GCH)]]
            pltpu.async_copy(src, buf.at[pl.ds(g * GCH, GCH)], sem)

    def drain(buf, sem):
        pltpu.make_async_copy(tab.at[pl.ds(0, CB * HIST)], buf, sem).wait()

    def reduce(c, buf):
        iota = lax.iota(_i32, 16)
        for b in range(CB):
            def body(j, acc):
                for u in range(_UNR):
                    r = b * HIST + j * _UNR + u
                    for h in range(2):
                        w = plsc.bitcast(buf[r, pl.ds(32 * h, 32)], _i32)
                        lo = plsc.bitcast(lax.shift_left(w, 16), _f32)
                        hi = plsc.bitcast(
                            lax.bitwise_and(w, jnp.int32(-65536)), _f32)
                        acc = (acc[:2 * h]
                               + (acc[2 * h] + lo, acc[2 * h + 1] + hi)
                               + acc[2 * h + 2:])
                return acc
            z = jnp.zeros((16,), _f32)
            a = lax.fori_loop(0, HIST // _UNR, body, (z, z, z, z))
            row = jnp.full((16,), c * CB + b, _i32)
            for h in range(2):
                plsc.store_scatter(sum_all, [row, 32 * h + 2 * iota],
                                   a[2 * h])
                plsc.store_scatter(sum_all, [row, 32 * h + 2 * iota + 1],
                                   a[2 * h + 1])

    fire(0, hb0, sem0)

    def pair(i, carry):
        c0 = 2 * i
        c1 = c0 + 1
        fire(c1, hb1, sem1)
        drain(hb0, sem0)
        reduce(c0, hb0)

        @pl.when(c1 + 1 < NCH)
        def _():
            fire(c1 + 1, hb0, sem0)

        drain(hb1, sem1)
        reduce(c1, hb1)
        return carry

    lax.fori_loop(0, NCH // 2, pair, 0)


def _sc_hist_body(idx_i, hidx_i, tab,
                  emb_o, sum_o,
                  idxv, hidxv, rows, hb0, hb1, ssum,
                  isem, hsem, usem, gsem0, gsem1, osem):
    wid = lax.axis_index("s") * NC + lax.axis_index("c")
    base = wid * PW
    b50 = base * HIST

    d_h = pltpu.async_copy(hidx_i.at[pl.ds(b50, PW * HIST)], hidxv, hsem)
    d_i = pltpu.async_copy(idx_i.at[pl.ds(base, PW)], idxv, isem)
    d_i.wait()
    g = pltpu.async_copy(tab.at[idxv], rows, usem)
    d_h.wait()
    _hist_pass(tab, hidxv, hb0, hb1, gsem0, gsem1, ssum)
    g.wait()
    dst = pl.ds(base, PW)
    o1 = pltpu.async_copy(rows, emb_o.at[dst], osem)
    o2 = pltpu.async_copy(ssum, sum_o.at[dst], osem)
    o1.wait()
    o2.wait()


def _sc_plain_body(idx_i, tab, emb_o, idxv, rows, isem, usem, osem):
    wid = lax.axis_index("s") * NC + lax.axis_index("c")
    base = wid * PW
    pltpu.async_copy(idx_i.at[pl.ds(base, PW)], idxv, isem).wait()
    pltpu.async_copy(tab.at[idxv], rows, usem).wait()
    pltpu.async_copy(rows, emb_o.at[pl.ds(base, PW)], osem).wait()


_SC_PARAMS = dict(
    mesh=plsc.VectorSubcoreMesh(core_axis_name="c", subcore_axis_name="s"),
    compiler_params=pltpu.CompilerParams(use_tc_tiling_on_sc=False,
                                         needs_layout_passes=False),
)

_sc_hist = functools.partial(
    pl.kernel,
    out_type=[jax.ShapeDtypeStruct((B, EDIM), _bf16),
              jax.ShapeDtypeStruct((B, EDIM), _f32)],
    scratch_types=[
        pltpu.VMEM((PW,), _i32),
        pltpu.VMEM((PW * HIST,), _i32),
        pltpu.VMEM((PW, EDIM), _bf16),
        pltpu.VMEM((CB * HIST, EDIM), _bf16),
        pltpu.VMEM((CB * HIST, EDIM), _bf16),
        pltpu.VMEM((PW, EDIM), _f32),
        pltpu.SemaphoreType.DMA,
        pltpu.SemaphoreType.DMA,
        pltpu.SemaphoreType.DMA,
        pltpu.SemaphoreType.DMA,
        pltpu.SemaphoreType.DMA,
        pltpu.SemaphoreType.DMA,
    ],
    **_SC_PARAMS,
)(_sc_hist_body)

_sc_plain = functools.partial(
    pl.kernel,
    out_type=jax.ShapeDtypeStruct((B, EDIM), _bf16),
    scratch_types=[
        pltpu.VMEM((PW,), _i32),
        pltpu.VMEM((PW, EDIM), _bf16),
        pltpu.SemaphoreType.DMA,
        pltpu.SemaphoreType.DMA,
        pltpu.SemaphoreType.DMA,
    ],
    **_SC_PARAMS,
)(_sc_plain_body)


_BT = 1024
_BN_SCALE = 1.0 / math.sqrt(1.0 + 1e-3)


def _mlp_body(u, m, c, ms, cs, g, bt, W1, b1, a1, W2, b2, a2, W3, b3, out):
    x = jnp.concatenate([u[...].astype(_f32), m[...].astype(_f32),
                         c[...].astype(_f32), ms[...], cs[...]], axis=1)
    x = g[...] * (x * _BN_SCALE) + bt[...]
    h = jnp.dot(x, W1[...], preferred_element_type=_f32) + b1[...]
    h = jnp.maximum(h, 0.0) + a1[...] * jnp.minimum(h, 0.0)
    h = jnp.dot(h, W2[...], preferred_element_type=_f32) + b2[...]
    h = jnp.maximum(h, 0.0) + a2[...] * jnp.minimum(h, 0.0)
    z = jnp.dot(h, W3[...], preferred_element_type=_f32) + b3[...]
    z = z - jnp.max(z, axis=1, keepdims=True)
    e = jnp.exp(z)
    out[...] = e / jnp.sum(e, axis=1, keepdims=True) + 1e-8


def _mlp(u, m, c, ms, cs, g, bt, W1, b1, a1, W2, b2, a2, W3, b3):
    emb_spec = pl.BlockSpec((_BT, EDIM), lambda i: (i, 0))
    def full(arr):
        return pl.BlockSpec(arr.shape, lambda i: (0,) * arr.ndim)
    return pl.pallas_call(
        _mlp_body,
        grid=(B // _BT,),
        in_specs=[emb_spec] * 5 + [full(a) for a in
                                   (g, bt, W1, b1, a1, W2, b2, a2, W3, b3)],
        out_specs=pl.BlockSpec((_BT, 2), lambda i: (i, 0)),
        out_shape=jax.ShapeDtypeStruct((B, 2), _f32),
    )(u, m, c, ms, cs, g, bt, W1, b1, a1, W2, b2, a2, W3, b3)


def kernel(uid_batch_ph, mid_batch_ph, cat_batch_ph, mid_his_batch_ph,
           cat_his_batch_ph, uid_table, mid_table, cat_table, bn_gamma,
           bn_beta, W1, b1, alpha1, W2, b2, alpha2, W3, b3):
    uid = uid_batch_ph.astype(_i32)
    mid = mid_batch_ph.astype(_i32)
    cat = cat_batch_ph.astype(_i32)
    mh = mid_his_batch_ph.astype(_i32).reshape(-1)
    ch = cat_his_batch_ph.astype(_i32).reshape(-1)
    ce, csum = _sc_hist(cat, ch, cat_table.astype(_bf16))
    me, msum = _sc_hist(mid, mh, mid_table.astype(_bf16))
    ue = _sc_plain(uid, uid_table.astype(_bf16))
    r2 = lambda a: a.reshape(1, -1)
    return _mlp(ue, me, ce, msum, csum, r2(bn_gamma), r2(bn_beta),
                W1, r2(b1), r2(alpha1), W2, r2(b2), r2(alpha2),
                W3, r2(b3))


# speedup vs baseline: 10.4633x; 1.2167x over previous
import functools
import math

import jax
import jax.numpy as jnp
from jax import lax
from jax.experimental import pallas as pl
from jax.experimental.pallas import tpu as pltpu
from jax.experimental.pallas import tpu_sc as plsc

B = 4096
HIST = 50
EDIM = 64
NC, NS = 2, 16
NW = NC * NS
PW = B // NW
CB = 8
NCH = PW // CB
GCH = 80
NG = (CB * HIST) // GCH
_UNR = 5

_f32 = jnp.float32
_bf16 = jnp.bfloat16
_i32 = jnp.int32


def _hist_pass(tab, idxv, hb0, hb1, sem0, sem1, sum_all):
    def fire(c, buf, sem):
        for g in range(NG):
            src = tab.at[idxv.at[pl.ds(c * CB * HIST + g * GCH, GCH)]]
            pltpu.async_copy(src, buf.at[pl.ds(g * GCH, GCH)], sem)

    def drain(buf, sem):
        pltpu.make_async_copy(tab.at[pl.ds(0, CB * HIST)], buf, sem).wait()

    def reduce(c, buf):
        for b in range(CB):
            def body(j, acc):
                for u in range(_UNR):
                    r = b * HIST + j * _UNR + u
                    acc = tuple(acc[d] + buf[r, pl.ds(16 * d, 16)]
                                for d in range(4))
                return acc
            z = jnp.zeros((16,), _f32)
            a = lax.fori_loop(0, HIST // _UNR, body, (z, z, z, z))
            row = c * CB + b
            for d in range(4):
                sum_all[row, pl.ds(16 * d, 16)] = a[d]

    fire(0, hb0, sem0)

    def pair(i, carry):
        c0 = 2 * i
        c1 = c0 + 1
        fire(c1, hb1, sem1)
        drain(hb0, sem0)
        reduce(c0, hb0)

        @pl.when(c1 + 1 < NCH)
        def _():
            fire(c1 + 1, hb0, sem0)

        drain(hb1, sem1)
        reduce(c1, hb1)
        return carry

    lax.fori_loop(0, NCH // 2, pair, 0)


def _sc_hist_body(idx_i, hidx_i, tab,
                  emb_o, sum_o,
                  idxv, hidxv, rows, hb0, hb1, ssum,
                  isem, hsem, usem, gsem0, gsem1, osem):
    wid = lax.axis_index("s") * NC + lax.axis_index("c")
    base = wid * PW
    b50 = base * HIST

    d_h = pltpu.async_copy(hidx_i.at[pl.ds(b50, PW * HIST)], hidxv, hsem)
    d_i = pltpu.async_copy(idx_i.at[pl.ds(base, PW)], idxv, isem)
    d_i.wait()
    g = pltpu.async_copy(tab.at[idxv], rows, usem)
    d_h.wait()
    _hist_pass(tab, hidxv, hb0, hb1, gsem0, gsem1, ssum)
    g.wait()
    dst = pl.ds(base, PW)
    o1 = pltpu.async_copy(rows, emb_o.at[dst], osem)
    o2 = pltpu.async_copy(ssum, sum_o.at[dst], osem)
    o1.wait()
    o2.wait()


def _sc_plain_body(idx_i, tab, emb_o, idxv, rows, isem, usem, osem):
    wid = lax.axis_index("s") * NC + lax.axis_index("c")
    base = wid * PW
    pltpu.async_copy(idx_i.at[pl.ds(base, PW)], idxv, isem).wait()
    pltpu.async_copy(tab.at[idxv], rows, usem).wait()
    pltpu.async_copy(rows, emb_o.at[pl.ds(base, PW)], osem).wait()


_SC_PARAMS = dict(
    mesh=plsc.VectorSubcoreMesh(core_axis_name="c", subcore_axis_name="s"),
    compiler_params=pltpu.CompilerParams(use_tc_tiling_on_sc=False,
                                         needs_layout_passes=False),
)

_sc_hist = functools.partial(
    pl.kernel,
    out_type=[jax.ShapeDtypeStruct((B, EDIM), _f32),
              jax.ShapeDtypeStruct((B, EDIM), _f32)],
    scratch_types=[
        pltpu.VMEM((PW,), _i32),
        pltpu.VMEM((PW * HIST,), _i32),
        pltpu.VMEM((PW, EDIM), _f32),
        pltpu.VMEM((CB * HIST, EDIM), _f32),
        pltpu.VMEM((CB * HIST, EDIM), _f32),
        pltpu.VMEM((PW, EDIM), _f32),
        pltpu.SemaphoreType.DMA,
        pltpu.SemaphoreType.DMA,
        pltpu.SemaphoreType.DMA,
        pltpu.SemaphoreType.DMA,
        pltpu.SemaphoreType.DMA,
        pltpu.SemaphoreType.DMA,
    ],
    **_SC_PARAMS,
)(_sc_hist_body)

_sc_plain = functools.partial(
    pl.kernel,
    out_type=jax.ShapeDtypeStruct((B, EDIM), _f32),
    scratch_types=[
        pltpu.VMEM((PW,), _i32),
        pltpu.VMEM((PW, EDIM), _f32),
        pltpu.SemaphoreType.DMA,
        pltpu.SemaphoreType.DMA,
        pltpu.SemaphoreType.DMA,
    ],
    **_SC_PARAMS,
)(_sc_plain_body)


_BT = 1024
_BN_SCALE = 1.0 / math.sqrt(1.0 + 1e-3)


def _mlp_body(u, m, c, ms, cs, g, bt, W1, b1, a1, W2, b2, a2, W3, b3, out):
    x = jnp.concatenate([u[...], m[...], c[...], ms[...], cs[...]], axis=1)
    x = g[...] * (x * _BN_SCALE) + bt[...]
    h = jnp.dot(x, W1[...], preferred_element_type=_f32) + b1[...]
    h = jnp.maximum(h, 0.0) + a1[...] * jnp.minimum(h, 0.0)
    h = jnp.dot(h, W2[...], preferred_element_type=_f32) + b2[...]
    h = jnp.maximum(h, 0.0) + a2[...] * jnp.minimum(h, 0.0)
    z = jnp.dot(h, W3[...], preferred_element_type=_f32) + b3[...]
    z = z - jnp.max(z, axis=1, keepdims=True)
    e = jnp.exp(z)
    out[...] = e / jnp.sum(e, axis=1, keepdims=True) + 1e-8


def _mlp(u, m, c, ms, cs, g, bt, W1, b1, a1, W2, b2, a2, W3, b3):
    emb_spec = pl.BlockSpec((_BT, EDIM), lambda i: (i, 0))
    def full(arr):
        return pl.BlockSpec(arr.shape, lambda i: (0,) * arr.ndim)
    return pl.pallas_call(
        _mlp_body,
        grid=(B // _BT,),
        in_specs=[emb_spec] * 5 + [full(a) for a in
                                   (g, bt, W1, b1, a1, W2, b2, a2, W3, b3)],
        out_specs=pl.BlockSpec((_BT, 2), lambda i: (i, 0)),
        out_shape=jax.ShapeDtypeStruct((B, 2), _f32),
    )(u, m, c, ms, cs, g, bt, W1, b1, a1, W2, b2, a2, W3, b3)


def kernel(uid_batch_ph, mid_batch_ph, cat_batch_ph, mid_his_batch_ph,
           cat_his_batch_ph, uid_table, mid_table, cat_table, bn_gamma,
           bn_beta, W1, b1, alpha1, W2, b2, alpha2, W3, b3):
    uid = uid_batch_ph.astype(_i32)
    mid = mid_batch_ph.astype(_i32)
    cat = cat_batch_ph.astype(_i32)
    mh = mid_his_batch_ph.astype(_i32).reshape(-1)
    ch = cat_his_batch_ph.astype(_i32).reshape(-1)
    ce, csum = _sc_hist(cat, ch, cat_table)
    me, msum = _sc_hist(mid, mh, mid_table)
    ue = _sc_plain(uid, uid_table)
    r2 = lambda a: a.reshape(1, -1)
    return _mlp(ue, me, ce, msum, csum, r2(bn_gamma), r2(bn_beta),
                W1, r2(b1), r2(alpha1), W2, r2(b2), r2(alpha2),
                W3, r2(b3))


# speedup vs baseline: 12.0649x; 1.1531x over previous
import functools
import math

import jax
import jax.numpy as jnp
from jax import lax
from jax.experimental import pallas as pl
from jax.experimental.pallas import tpu as pltpu
from jax.experimental.pallas import tpu_sc as plsc

B = 4096
HIST = 50
EDIM = 64
N_UID = 100000
NC, NS = 2, 16
NW = NC * NS
PW = B // NW
CB = 8
NCH = PW // CB
GCH = 80
NG = (CB * HIST) // GCH
_UNR = 5

_f32 = jnp.float32
_i32 = jnp.int32


def _hist_pass(tab, idxv, hb0, hb1, sem0, sem1, sum_all):
    def fire(c, buf, sem):
        for g in range(NG):
            src = tab.at[idxv.at[pl.ds(c * CB * HIST + g * GCH, GCH)]]
            pltpu.async_copy(src, buf.at[pl.ds(g * GCH, GCH)], sem)

    def drain(buf, sem):
        pltpu.make_async_copy(tab.at[pl.ds(0, CB * HIST)], buf, sem).wait()

    def reduce(c, buf):
        for b in range(CB):
            def body(j, acc):
                for u in range(_UNR):
                    r = b * HIST + j * _UNR + u
                    acc = tuple(acc[d] + buf[r, pl.ds(16 * d, 16)]
                                for d in range(4))
                return acc
            z = jnp.zeros((16,), _f32)
            a = lax.fori_loop(0, HIST // _UNR, body, (z, z, z, z))
            row = c * CB + b
            for d in range(4):
                sum_all[row, pl.ds(16 * d, 16)] = a[d]

    fire(0, hb0, sem0)

    def pair(i, carry):
        c0 = 2 * i
        c1 = c0 + 1
        fire(c1, hb1, sem1)
        drain(hb0, sem0)
        reduce(c0, hb0)

        @pl.when(c1 + 1 < NCH)
        def _():
            fire(c1 + 1, hb0, sem0)

        drain(hb1, sem1)
        reduce(c1, hb1)
        return carry

    lax.fori_loop(0, NCH // 2, pair, 0)


def _sc_hist_body(idx_i, hidx_i, tab,
                  emb_o, sum_o,
                  idxv, hidxv, rows, hb0, hb1, ssum,
                  isem, hsem, usem, gsem0, gsem1, osem):
    wid = lax.axis_index("s") * NC + lax.axis_index("c")
    base = wid * PW
    b50 = base * HIST

    d_h = pltpu.async_copy(hidx_i.at[pl.ds(b50, PW * HIST)], hidxv, hsem)
    d_i = pltpu.async_copy(idx_i.at[pl.ds(base, PW)], idxv, isem)
    d_i.wait()
    g = pltpu.async_copy(tab.at[idxv], rows, usem)
    d_h.wait()
    _hist_pass(tab, hidxv, hb0, hb1, gsem0, gsem1, ssum)
    g.wait()
    dst = pl.ds(base, PW)
    o1 = pltpu.async_copy(rows, emb_o.at[dst], osem)
    o2 = pltpu.async_copy(ssum, sum_o.at[dst], osem)
    o1.wait()
    o2.wait()


def _sc_flat_body(uid_i, ut, ue_o, uidv, idxb, rowsf, isem, gsem, osem):
    wid = lax.axis_index("s") * NC + lax.axis_index("c")
    base = wid * PW
    pltpu.async_copy(uid_i.at[pl.ds(base, PW)], uidv, isem).wait()

    foffs = [(lax.iota(_i32, 16) + 16 * k) * N_UID for k in range(4)]

    def build(c, carry):
        for b in range(CB):
            r = c * CB + b
            bc = plsc.load_gather(uidv, [jnp.full((16,), r, _i32)])
            for k in range(4):
                idxb[pl.ds(r * EDIM + 16 * k, 16)] = bc + foffs[k]
        return carry

    lax.fori_loop(0, NCH, build, 0)

    def fire(d, carry):
        src = ut.at[idxb.at[pl.ds(d * 128, 128)]]
        pltpu.async_copy(src, rowsf.at[pl.ds(d * 128, 128)], gsem)
        return carry

    lax.fori_loop(0, PW * EDIM // 128, fire, 0)
    pltpu.make_async_copy(ut.at[pl.ds(0, PW * EDIM)], rowsf, gsem).wait()
    pltpu.async_copy(rowsf, ue_o.at[pl.ds(base * EDIM, PW * EDIM)],
                     osem).wait()


_SC_PARAMS = dict(
    mesh=plsc.VectorSubcoreMesh(core_axis_name="c", subcore_axis_name="s"),
    compiler_params=pltpu.CompilerParams(use_tc_tiling_on_sc=False,
                                         needs_layout_passes=False),
)

_sc_hist = functools.partial(
    pl.kernel,
    out_type=[jax.ShapeDtypeStruct((B, EDIM), _f32),
              jax.ShapeDtypeStruct((B, EDIM), _f32)],
    scratch_types=[
        pltpu.VMEM((PW,), _i32),
        pltpu.VMEM((PW * HIST,), _i32),
        pltpu.VMEM((PW, EDIM), _f32),
        pltpu.VMEM((CB * HIST, EDIM), _f32),
        pltpu.VMEM((CB * HIST, EDIM), _f32),
        pltpu.VMEM((PW, EDIM), _f32),
        pltpu.SemaphoreType.DMA,
        pltpu.SemaphoreType.DMA,
        pltpu.SemaphoreType.DMA,
        pltpu.SemaphoreType.DMA,
        pltpu.SemaphoreType.DMA,
        pltpu.SemaphoreType.DMA,
    ],
    **_SC_PARAMS,
)(_sc_hist_body)

_sc_flat = functools.partial(
    pl.kernel,
    out_type=jax.ShapeDtypeStruct((B * EDIM,), _f32),
    scratch_types=[
        pltpu.VMEM((PW,), _i32),
        pltpu.VMEM((PW * EDIM,), _i32),
        pltpu.VMEM((PW * EDIM,), _f32),
        pltpu.SemaphoreType.DMA,
        pltpu.SemaphoreType.DMA,
        pltpu.SemaphoreType.DMA,
    ],
    **_SC_PARAMS,
)(_sc_flat_body)


_BT = 1024
_BN_SCALE = 1.0 / math.sqrt(1.0 + 1e-3)


def _mlp_body(u, m, c, ms, cs, g, bt, W1, b1, a1, W2, b2, a2, W3, b3, out):
    x = jnp.concatenate([u[...], m[...], c[...], ms[...], cs[...]], axis=1)
    x = g[...] * (x * _BN_SCALE) + bt[...]
    h = jnp.dot(x, W1[...], preferred_element_type=_f32) + b1[...]
    h = jnp.maximum(h, 0.0) + a1[...] * jnp.minimum(h, 0.0)
    h = jnp.dot(h, W2[...], preferred_element_type=_f32) + b2[...]
    h = jnp.maximum(h, 0.0) + a2[...] * jnp.minimum(h, 0.0)
    z = jnp.dot(h, W3[...], preferred_element_type=_f32) + b3[...]
    z = z - jnp.max(z, axis=1, keepdims=True)
    e = jnp.exp(z)
    out[...] = e / jnp.sum(e, axis=1, keepdims=True) + 1e-8


def _mlp(u, m, c, ms, cs, g, bt, W1, b1, a1, W2, b2, a2, W3, b3):
    emb_spec = pl.BlockSpec((_BT, EDIM), lambda i: (i, 0))
    def full(arr):
        return pl.BlockSpec(arr.shape, lambda i: (0,) * arr.ndim)
    return pl.pallas_call(
        _mlp_body,
        grid=(B // _BT,),
        in_specs=[emb_spec] * 5 + [full(a) for a in
                                   (g, bt, W1, b1, a1, W2, b2, a2, W3, b3)],
        out_specs=pl.BlockSpec((_BT, 2), lambda i: (i, 0)),
        out_shape=jax.ShapeDtypeStruct((B, 2), _f32),
    )(u, m, c, ms, cs, g, bt, W1, b1, a1, W2, b2, a2, W3, b3)


def kernel(uid_batch_ph, mid_batch_ph, cat_batch_ph, mid_his_batch_ph,
           cat_his_batch_ph, uid_table, mid_table, cat_table, bn_gamma,
           bn_beta, W1, b1, alpha1, W2, b2, alpha2, W3, b3):
    uid = uid_batch_ph.astype(_i32)
    mid = mid_batch_ph.astype(_i32)
    cat = cat_batch_ph.astype(_i32)
    mh = mid_his_batch_ph.astype(_i32).reshape(-1)
    ch = cat_his_batch_ph.astype(_i32).reshape(-1)
    ce, csum = _sc_hist(cat, ch, cat_table)
    me, msum = _sc_hist(mid, mh, mid_table)
    ue = _sc_flat(uid, uid_table.T.reshape(-1)).reshape(B, EDIM)
    r2 = lambda a: a.reshape(1, -1)
    return _mlp(ue, me, ce, msum, csum, r2(bn_gamma), r2(bn_beta),
                W1, r2(b1), r2(alpha1), W2, r2(b2), r2(alpha2),
                W3, r2(b3))


# speedup vs baseline: 12.3585x; 1.0243x over previous
import functools
import math

import jax
import jax.numpy as jnp
from jax import lax
from jax.experimental import pallas as pl
from jax.experimental.pallas import tpu as pltpu
from jax.experimental.pallas import tpu_sc as plsc

B = 4096
HIST = 50
EDIM = 64
N_UID = 100000
N_CAT = 1000
NC, NS = 2, 16
NW = NC * NS
PW = B // NW
CB = 8
NCH = PW // CB
GCH = 80
NG = (CB * HIST) // GCH
_UNR = 5

_f32 = jnp.float32
_i32 = jnp.int32


def _hist_pass(tab, idxv, hb0, hb1, sem0, sem1, sum_all):
    def fire(c, buf, sem):
        for g in range(NG):
            src = tab.at[idxv.at[pl.ds(c * CB * HIST + g * GCH, GCH)]]
            pltpu.async_copy(src, buf.at[pl.ds(g * GCH, GCH)], sem)

    def drain(buf, sem):
        pltpu.make_async_copy(tab.at[pl.ds(0, CB * HIST)], buf, sem).wait()

    def reduce(c, buf):
        for b in range(CB):
            def body(j, acc):
                for u in range(_UNR):
                    r = b * HIST + j * _UNR + u
                    acc = tuple(acc[d] + buf[r, pl.ds(16 * d, 16)]
                                for d in range(4))
                return acc
            z = jnp.zeros((16,), _f32)
            a = lax.fori_loop(0, HIST // _UNR, body, (z, z, z, z))
            row = c * CB + b
            for d in range(4):
                sum_all[row, pl.ds(16 * d, 16)] = a[d]

    fire(0, hb0, sem0)

    def pair(i, carry):
        c0 = 2 * i
        c1 = c0 + 1
        fire(c1, hb1, sem1)
        drain(hb0, sem0)
        reduce(c0, hb0)

        @pl.when(c1 + 1 < NCH)
        def _():
            fire(c1 + 1, hb0, sem0)

        drain(hb1, sem1)
        reduce(c1, hb1)
        return carry

    lax.fori_loop(0, NCH // 2, pair, 0)


def _sc_hist_body(idx_i, hidx_i, tab,
                  emb_o, sum_o,
                  idxv, hidxv, rows, hb0, hb1, ssum,
                  isem, hsem, usem, gsem0, gsem1, osem):
    wid = lax.axis_index("s") * NC + lax.axis_index("c")
    base = wid * PW
    b50 = base * HIST

    d_h = pltpu.async_copy(hidx_i.at[pl.ds(b50, PW * HIST)], hidxv, hsem)
    d_i = pltpu.async_copy(idx_i.at[pl.ds(base, PW)], idxv, isem)
    d_i.wait()
    g = pltpu.async_copy(tab.at[idxv], rows, usem)
    d_h.wait()
    _hist_pass(tab, hidxv, hb0, hb1, gsem0, gsem1, ssum)
    g.wait()
    dst = pl.ds(base, PW)
    o1 = pltpu.async_copy(rows, emb_o.at[dst], osem)
    o2 = pltpu.async_copy(ssum, sum_o.at[dst], osem)
    o1.wait()
    o2.wait()


def _sc_flat_body(uid_i, ut, ue_o, uidv, idxb, rowsf, isem, gsem, osem):
    wid = lax.axis_index("s") * NC + lax.axis_index("c")
    base = wid * PW
    pltpu.async_copy(uid_i.at[pl.ds(base, PW)], uidv, isem).wait()

    foffs = [(lax.iota(_i32, 16) + 16 * k) * N_UID for k in range(4)]

    def build(c, carry):
        for b in range(CB):
            r = c * CB + b
            bc = plsc.load_gather(uidv, [jnp.full((16,), r, _i32)])
            for k in range(4):
                idxb[pl.ds(r * EDIM + 16 * k, 16)] = bc + foffs[k]
        return carry

    lax.fori_loop(0, NCH, build, 0)

    def fire(d, carry):
        src = ut.at[idxb.at[pl.ds(d * 128, 128)]]
        pltpu.async_copy(src, rowsf.at[pl.ds(d * 128, 128)], gsem)
        return carry

    lax.fori_loop(0, PW * EDIM // 128, fire, 0)
    pltpu.make_async_copy(ut.at[pl.ds(0, PW * EDIM)], rowsf, gsem).wait()
    pltpu.async_copy(rowsf, ue_o.at[pl.ds(base * EDIM, PW * EDIM)],
                     osem).wait()


_SC_PARAMS = dict(
    mesh=plsc.VectorSubcoreMesh(core_axis_name="c", subcore_axis_name="s"),
    compiler_params=pltpu.CompilerParams(use_tc_tiling_on_sc=False,
                                         needs_layout_passes=False),
)

_sc_hist = functools.partial(
    pl.kernel,
    out_type=[jax.ShapeDtypeStruct((B, EDIM), _f32),
              jax.ShapeDtypeStruct((B, EDIM), _f32)],
    scratch_types=[
        pltpu.VMEM((PW,), _i32),
        pltpu.VMEM((PW * HIST,), _i32),
        pltpu.VMEM((PW, EDIM), _f32),
        pltpu.VMEM((CB * HIST, EDIM), _f32),
        pltpu.VMEM((CB * HIST, EDIM), _f32),
        pltpu.VMEM((PW, EDIM), _f32),
        pltpu.SemaphoreType.DMA,
        pltpu.SemaphoreType.DMA,
        pltpu.SemaphoreType.DMA,
        pltpu.SemaphoreType.DMA,
        pltpu.SemaphoreType.DMA,
        pltpu.SemaphoreType.DMA,
    ],
    **_SC_PARAMS,
)(_sc_hist_body)

_sc_flat = functools.partial(
    pl.kernel,
    out_type=jax.ShapeDtypeStruct((B * EDIM,), _f32),
    scratch_types=[
        pltpu.VMEM((PW,), _i32),
        pltpu.VMEM((PW * EDIM,), _i32),
        pltpu.VMEM((PW * EDIM,), _f32),
        pltpu.SemaphoreType.DMA,
        pltpu.SemaphoreType.DMA,
        pltpu.SemaphoreType.DMA,
    ],
    **_SC_PARAMS,
)(_sc_flat_body)


_BT = 1024
_BN_SCALE = 1.0 / math.sqrt(1.0 + 1e-3)


def _mlp_body(u, m, c, ms, cs, g, bt, W1, b1, a1, W2, b2, a2, W3, b3, out):
    x = jnp.concatenate([u[...], m[...], c[...], ms[...], cs[...]], axis=1)
    x = g[...] * (x * _BN_SCALE) + bt[...]
    h = jnp.dot(x, W1[...], preferred_element_type=_f32) + b1[...]
    h = jnp.maximum(h, 0.0) + a1[...] * jnp.minimum(h, 0.0)
    h = jnp.dot(h, W2[...], preferred_element_type=_f32) + b2[...]
    h = jnp.maximum(h, 0.0) + a2[...] * jnp.minimum(h, 0.0)
    z = jnp.dot(h, W3[...], preferred_element_type=_f32) + b3[...]
    z = z - jnp.max(z, axis=1, keepdims=True)
    e = jnp.exp(z)
    out[...] = e / jnp.sum(e, axis=1, keepdims=True) + 1e-8


def _mlp(u, m, c, ms, cs, g, bt, W1, b1, a1, W2, b2, a2, W3, b3):
    emb_spec = pl.BlockSpec((_BT, EDIM), lambda i: (i, 0))
    def full(arr):
        return pl.BlockSpec(arr.shape, lambda i: (0,) * arr.ndim)
    return pl.pallas_call(
        _mlp_body,
        grid=(B // _BT,),
        in_specs=[emb_spec] * 5 + [full(a) for a in
                                   (g, bt, W1, b1, a1, W2, b2, a2, W3, b3)],
        out_specs=pl.BlockSpec((_BT, 2), lambda i: (i, 0)),
        out_shape=jax.ShapeDtypeStruct((B, 2), _f32),
    )(u, m, c, ms, cs, g, bt, W1, b1, a1, W2, b2, a2, W3, b3)


def kernel(uid_batch_ph, mid_batch_ph, cat_batch_ph, mid_his_batch_ph,
           cat_his_batch_ph, uid_table, mid_table, cat_table, bn_gamma,
           bn_beta, W1, b1, alpha1, W2, b2, alpha2, W3, b3):
    uid = uid_batch_ph.astype(_i32)
    mid = mid_batch_ph.astype(_i32)
    cat = cat_batch_ph.astype(_i32)
    mh = mid_his_batch_ph.astype(_i32).reshape(-1)
    ch = cat_his_batch_ph.astype(_i32).reshape(-1)
    rep = 16
    cat_rep = jnp.tile(cat_table, (rep, 1))
    ch = ch + (jnp.arange(B * HIST, dtype=_i32) % rep) * N_CAT
    ce, csum = _sc_hist(cat, ch, cat_rep)
    me, msum = _sc_hist(mid, mh, mid_table)
    ue = _sc_flat(uid, uid_table.T.reshape(-1)).reshape(B, EDIM)
    r2 = lambda a: a.reshape(1, -1)
    return _mlp(ue, me, ce, msum, csum, r2(bn_gamma), r2(bn_beta),
                W1, r2(b1), r2(alpha1), W2, r2(b2), r2(alpha2),
                W3, r2(b3))
